# Initial kernel scaffold; baseline (speedup 1.0000x reference)
import jax
import jax.numpy as jnp
from jax.experimental import pallas as pl


def kernel(inputs1, inputs2, neg, neighbors, feat_data, degrees, W1, b1, W2, b2):
    raise NotImplementedError("write your pallas kernel here")


# speedup vs baseline: 4.4831x; 4.4831x over previous
import functools

import jax
import jax.numpy as jnp
from jax import lax
from jax.experimental import pallas as pl
from jax.experimental.pallas import tpu as pltpu
from jax.experimental.pallas import tpu_sc as plsc

N = 100000
D = 128
DEG = 16
B = 1024
NEG = 10
Q = 2 * B + NEG
NW = 32
QP = 2304
BPW = QP // NW
CH = 24
NCH = BPW // CH


def _sc_gather_kernel(nodes_hbm, neighbors_hbm, feat_hbm, deg16_hbm,
                      s_out, degr_out, nf_out,
                      nodes_v, nb_v, nb1d, idxd, degrows_v, degv, rows_v,
                      nf_v, s_v, sem):
    wid = lax.axis_index("s") * 2 + lax.axis_index("c")
    base = wid * BPW

    pltpu.sync_copy(nodes_hbm.at[pl.ds(base, BPW)], nodes_v)
    pltpu.async_copy(neighbors_hbm.at[nodes_v], nb_v, sem).wait()

    def flatten_row(q, _):
        v = nb_v[q, :]
        nb1d[pl.ds(q * DEG, DEG)] = v
        idxd[pl.ds(q * DEG, DEG)] = lax.shift_right_logical(v, 4)
        return 0

    lax.fori_loop(0, BPW, flatten_row, 0)

    pltpu.async_copy(feat_hbm.at[nodes_v], nf_v, sem).wait()
    pltpu.sync_copy(nf_v, nf_out.at[pl.ds(base, BPW)])

    pltpu.async_copy(deg16_hbm.at[idxd], degrows_v, sem).wait()
    lanes = lax.iota(jnp.int32, 16)

    def pick_deg(q, _):
        v = nb_v[q, :]
        col = lax.bitwise_and(v, 15)
        row = q * DEG + lanes
        degv[pl.ds(q * DEG, DEG)] = plsc.load_gather(degrows_v, [row, col])
        return 0

    lax.fori_loop(0, BPW, pick_deg, 0)
    pltpu.sync_copy(degv, degr_out.at[pl.ds(base * DEG, BPW * DEG)])

    for ci in range(NCH):
        pltpu.async_copy(feat_hbm.at[nb1d.at[pl.ds(ci * CH * DEG, CH * DEG)]],
                         rows_v, sem).wait()

        def reduce_one(q, _):
            for c in range(D // 16):
                acc = rows_v[q * DEG, pl.ds(c * 16, 16)]
                for k in range(1, DEG):
                    acc = acc + rows_v[q * DEG + k, pl.ds(c * 16, 16)]
                s_v[ci * CH + q, pl.ds(c * 16, 16)] = acc
            return 0

        lax.fori_loop(0, CH, reduce_one, 0)

    pltpu.sync_copy(s_v, s_out.at[pl.ds(base, BPW)])


@functools.lru_cache(maxsize=None)
def _build_sc_gather():
    return pl.kernel(
        _sc_gather_kernel,
        out_type=[
            jax.ShapeDtypeStruct((QP, D), jnp.float32),
            jax.ShapeDtypeStruct((QP * DEG,), jnp.float32),
            jax.ShapeDtypeStruct((QP, D), jnp.float32),
        ],
        mesh=plsc.VectorSubcoreMesh(core_axis_name="c",
                                    subcore_axis_name="s"),
        compiler_params=pltpu.CompilerParams(use_tc_tiling_on_sc=False,
                                             needs_layout_passes=False),
        scratch_types=[
            pltpu.VMEM((BPW,), jnp.int32),
            pltpu.VMEM((BPW, DEG), jnp.int32),
            pltpu.VMEM((BPW * DEG,), jnp.int32),
            pltpu.VMEM((BPW * DEG,), jnp.int32),
            pltpu.VMEM((BPW * DEG, DEG), jnp.float32),
            pltpu.VMEM((BPW * DEG,), jnp.float32),
            pltpu.VMEM((CH * DEG, D), jnp.float32),
            pltpu.VMEM((BPW, D), jnp.float32),
            pltpu.VMEM((BPW, D), jnp.float32),
            pltpu.SemaphoreType.DMA,
        ],
    )


def _tc_tail_kernel(s_ref, deg_ref, nf_ref, w1_ref, b1_ref, w2_ref, b2_ref,
                    o_ref):
    g = jnp.sum(deg_ref[...], axis=1, keepdims=True)
    w1s = jnp.sum(w1_ref[:, D:], axis=1)
    ne = lax.dot_general(s_ref[...], w1_ref[:, :D],
                         (((1,), (1,)), ((), ())),
                         preferred_element_type=jnp.float32)
    ne = ne + g * w1s[None, :] + float(DEG) * b1_ref[...]
    f = lax.dot_general(nf_ref[...], w2_ref[:, :D],
                        (((1,), (1,)), ((), ())),
                        preferred_element_type=jnp.float32)
    f = f + lax.dot_general(ne, w2_ref[:, D:],
                            (((1,), (1,)), ((), ())),
                            preferred_element_type=jnp.float32)
    f = f + b2_ref[...]
    n = jnp.sqrt(jnp.sum(f * f, axis=1, keepdims=True))
    o_ref[...] = f / jnp.maximum(n, 1e-12)


def kernel(inputs1, inputs2, neg, neighbors, feat_data, degrees, W1, b1, W2,
           b2):
    nodes = jnp.concatenate([inputs1, inputs2, neg]).astype(jnp.int32)
    nodes = jnp.pad(nodes, (0, QP - Q))
    deg16 = degrees.reshape(N // DEG, DEG)

    s, degr, nf = _build_sc_gather()(nodes, neighbors.astype(jnp.int32),
                                     feat_data, deg16)

    out = pl.pallas_call(
        _tc_tail_kernel,
        out_shape=jax.ShapeDtypeStruct((QP, D), jnp.float32),
    )(s, degr.reshape(QP, DEG), nf, W1, b1.reshape(1, D), W2,
      b2.reshape(1, D))

    return out[:B], out[B:2 * B], out[2 * B:Q]


# speedup vs baseline: 4.7250x; 1.0540x over previous
import functools

import jax
import jax.numpy as jnp
from jax import lax
from jax.experimental import pallas as pl
from jax.experimental.pallas import tpu as pltpu
from jax.experimental.pallas import tpu_sc as plsc

N = 100000
D = 128
DEG = 16
B = 1024
NEG = 10
Q = 2 * B + NEG
NW = 32
QP = 2304
BPW = QP // NW
CH = 18
NCH = BPW // CH


def _sc_gather_kernel(nodes_hbm, neighbors_hbm, feat_hbm, deg16_hbm,
                      s_out, degr_out, nf_out,
                      nodes_v, nb_v, nb1d, idxd, degrows_v, degv, rows_a,
                      rows_b, nf_v, s_v, sem_nf, sem_deg, sem_a, sem_b):
    wid = lax.axis_index("s") * 2 + lax.axis_index("c")
    base = wid * BPW

    pltpu.sync_copy(nodes_hbm.at[pl.ds(base, BPW)], nodes_v)
    pltpu.async_copy(neighbors_hbm.at[nodes_v], nb_v, sem_a).wait()

    def flatten_row(q, _):
        v = nb_v[q, :]
        nb1d[pl.ds(q * DEG, DEG)] = v
        idxd[pl.ds(q * DEG, DEG)] = lax.shift_right_logical(v, 4)
        return 0

    lax.fori_loop(0, BPW, flatten_row, 0)

    rows = (rows_a, rows_b)
    sems = (sem_a, sem_b)

    def start(ci):
        return pltpu.async_copy(
            feat_hbm.at[nb1d.at[pl.ds(ci * CH * DEG, CH * DEG)]],
            rows[ci % 2], sems[ci % 2])

    cur = start(0)
    nf_copy = pltpu.async_copy(feat_hbm.at[nodes_v], nf_v, sem_nf)
    deg_copy = pltpu.async_copy(deg16_hbm.at[idxd], degrows_v, sem_deg)
    for ci in range(NCH):
        nxt = start(ci + 1) if ci + 1 < NCH else None
        cur.wait()
        rv = rows[ci % 2]

        def reduce_one(q, _):
            for c in range(D // 16):
                acc = rv[q * DEG, pl.ds(c * 16, 16)]
                for k in range(1, DEG):
                    acc = acc + rv[q * DEG + k, pl.ds(c * 16, 16)]
                s_v[ci * CH + q, pl.ds(c * 16, 16)] = acc
            return 0

        lax.fori_loop(0, CH, reduce_one, 0)
        cur = nxt

    pltpu.sync_copy(s_v, s_out.at[pl.ds(base, BPW)])

    nf_copy.wait()
    pltpu.sync_copy(nf_v, nf_out.at[pl.ds(base, BPW)])

    deg_copy.wait()
    lanes = lax.iota(jnp.int32, 16)

    def pick_deg(q, _):
        v = nb_v[q, :]
        col = lax.bitwise_and(v, 15)
        row = q * DEG + lanes
        degv[pl.ds(q * DEG, DEG)] = plsc.load_gather(degrows_v, [row, col])
        return 0

    lax.fori_loop(0, BPW, pick_deg, 0)
    pltpu.sync_copy(degv, degr_out.at[pl.ds(base * DEG, BPW * DEG)])


@functools.lru_cache(maxsize=None)
def _build_sc_gather():
    return pl.kernel(
        _sc_gather_kernel,
        out_type=[
            jax.ShapeDtypeStruct((QP, D), jnp.float32),
            jax.ShapeDtypeStruct((QP * DEG,), jnp.float32),
            jax.ShapeDtypeStruct((QP, D), jnp.float32),
        ],
        mesh=plsc.VectorSubcoreMesh(core_axis_name="c",
                                    subcore_axis_name="s"),
        compiler_params=pltpu.CompilerParams(use_tc_tiling_on_sc=False,
                                             needs_layout_passes=False),
        scratch_types=[
            pltpu.VMEM((BPW,), jnp.int32),
            pltpu.VMEM((BPW, DEG), jnp.int32),
            pltpu.VMEM((BPW * DEG,), jnp.int32),
            pltpu.VMEM((BPW * DEG,), jnp.int32),
            pltpu.VMEM((BPW * DEG, DEG), jnp.float32),
            pltpu.VMEM((BPW * DEG,), jnp.float32),
            pltpu.VMEM((CH * DEG, D), jnp.float32),
            pltpu.VMEM((CH * DEG, D), jnp.float32),
            pltpu.VMEM((BPW, D), jnp.float32),
            pltpu.VMEM((BPW, D), jnp.float32),
            pltpu.SemaphoreType.DMA,
            pltpu.SemaphoreType.DMA,
            pltpu.SemaphoreType.DMA,
            pltpu.SemaphoreType.DMA,
        ],
    )


def _tc_tail_kernel(s_ref, deg_ref, nf_ref, w1_ref, b1_ref, w2_ref, b2_ref,
                    o_ref):
    g = jnp.sum(deg_ref[...], axis=1, keepdims=True)
    w1s = jnp.sum(w1_ref[:, D:], axis=1)
    ne = lax.dot_general(s_ref[...], w1_ref[:, :D],
                         (((1,), (1,)), ((), ())),
                         preferred_element_type=jnp.float32)
    ne = ne + g * w1s[None, :] + float(DEG) * b1_ref[...]
    f = lax.dot_general(nf_ref[...], w2_ref[:, :D],
                        (((1,), (1,)), ((), ())),
                        preferred_element_type=jnp.float32)
    f = f + lax.dot_general(ne, w2_ref[:, D:],
                            (((1,), (1,)), ((), ())),
                            preferred_element_type=jnp.float32)
    f = f + b2_ref[...]
    n = jnp.sqrt(jnp.sum(f * f, axis=1, keepdims=True))
    o_ref[...] = f / jnp.maximum(n, 1e-12)


def kernel(inputs1, inputs2, neg, neighbors, feat_data, degrees, W1, b1, W2,
           b2):
    nodes = jnp.concatenate([inputs1, inputs2, neg]).astype(jnp.int32)
    nodes = jnp.pad(nodes, (0, QP - Q))
    deg16 = degrees.reshape(N // DEG, DEG)

    s, degr, nf = _build_sc_gather()(nodes, neighbors.astype(jnp.int32),
                                     feat_data, deg16)

    out = pl.pallas_call(
        _tc_tail_kernel,
        out_shape=jax.ShapeDtypeStruct((QP, D), jnp.float32),
    )(s, degr.reshape(QP, DEG), nf, W1, b1.reshape(1, D), W2,
      b2.reshape(1, D))

    return out[:B], out[B:2 * B], out[2 * B:Q]
